# baseline (speedup 1.0000x reference)
import functools

import jax
import jax.numpy as jnp
from jax import lax
from jax.experimental import pallas as pl
from jax.experimental.pallas import tpu as pltpu
from jax.experimental.pallas import tpu_sc as plsc

N = 10000
E = 320000
D_IN = 128
D_H = 128
D_OUT = 40

NC = 2
NS = 16
NW = NC * NS
L = 16

NPAD = 10240
E_PER_W = E // NW


def _sc_degrees(src, dst):
    CH = 2000
    NT = NPAD // NS

    @functools.partial(
        pl.kernel,
        out_type=jax.ShapeDtypeStruct((NC, 2, NPAD), jnp.float32),
        mesh=plsc.VectorSubcoreMesh(core_axis_name="c", subcore_axis_name="s"),
        scratch_types=[
            pltpu.VMEM((CH,), jnp.int32),
            pltpu.VMEM((CH,), jnp.float32),
            pltpu.VMEM((NT,), jnp.float32),
            pltpu.VMEM_SHARED((NPAD,), jnp.float32),
            pltpu.VMEM_SHARED((NPAD,), jnp.float32),
        ],
    )
    def k(src_hbm, dst_hbm, out_hbm, idxv, onesv, tmpv, acc_s, acc_d):
        cid = lax.axis_index("c")
        sid = lax.axis_index("s")
        wid = cid * NS + sid

        def fill(i, _):
            onesv[pl.ds(i * L, L)] = jnp.full((L,), 1.0, jnp.float32)
            tmpv[pl.ds((i % (NT // L)) * L, L)] = jnp.zeros((L,), jnp.float32)
            return 0

        lax.fori_loop(0, CH // L, fill, 0)

        pltpu.sync_copy(tmpv, acc_s.at[pl.ds(sid * NT, NT)])
        pltpu.sync_copy(tmpv, acc_d.at[pl.ds(sid * NT, NT)])
        plsc.subcore_barrier()

        ebase = wid * E_PER_W

        def chunk(c, _):
            base = ebase + c * CH
            pltpu.sync_copy(src_hbm.at[pl.ds(base, CH)], idxv)
            pltpu.sync_copy(onesv, acc_s.at[idxv], add=True)
            pltpu.sync_copy(dst_hbm.at[pl.ds(base, CH)], idxv)
            pltpu.sync_copy(onesv, acc_d.at[idxv], add=True)
            return 0

        lax.fori_loop(0, E_PER_W // CH, chunk, 0)
        plsc.subcore_barrier()

        pltpu.sync_copy(acc_s.at[pl.ds(sid * NT, NT)], tmpv)
        pltpu.sync_copy(tmpv, out_hbm.at[cid, 0, pl.ds(sid * NT, NT)])
        pltpu.sync_copy(acc_d.at[pl.ds(sid * NT, NT)], tmpv)
        pltpu.sync_copy(tmpv, out_hbm.at[cid, 1, pl.ds(sid * NT, NT)])

    return k(src, dst)


def _sc_spmm(h, src3, dst3, d):
    CH = 128
    RT = NPAD // NS
    GCH = 16
    G = 5
    WB = RT // CH

    @functools.partial(
        pl.kernel,
        out_type=jax.ShapeDtypeStruct((NC, NPAD, d), jnp.float32),
        mesh=plsc.VectorSubcoreMesh(core_axis_name="c", subcore_axis_name="s"),
        scratch_types=[
            pltpu.VMEM((GCH, CH), jnp.int32),
            pltpu.VMEM((GCH, CH), jnp.int32),
            pltpu.VMEM((GCH, CH), jnp.int32),
            pltpu.VMEM((GCH, CH), jnp.int32),
            pltpu.VMEM((CH, d), jnp.float32),
            pltpu.VMEM((CH, d), jnp.float32),
            pltpu.VMEM_SHARED((NPAD, d), jnp.float32),
            pltpu.SemaphoreType.DMA,
            pltpu.SemaphoreType.DMA,
            pltpu.SemaphoreType.DMA,
            pltpu.SemaphoreType.DMA,
            pltpu.SemaphoreType.DMA,
        ],
    )
    def k(h_hbm, src_hbm, dst_hbm, out_hbm,
          sbufa, dbufa, sbufb, dbufb, rows0, rows1, acc,
          semi, semg0, semg1, sems0, sems1):
        cid = lax.axis_index("c")
        sid = lax.axis_index("s")
        wid = cid * NS + sid

        def drain(rows, sems):
            pltpu.make_async_copy(rows, acc.at[pl.ds(0, CH)], sems).wait()

        pltpu.async_copy(src_hbm.at[wid, pl.ds(0, GCH)], sbufa, semi)
        pltpu.async_copy(dst_hbm.at[wid, pl.ds(0, GCH)], dbufa, semi)

        def zfill(i, _):
            r = i // (d // L)
            c = i % (d // L)
            rows0[r, pl.ds(c * L, L)] = jnp.zeros((L,), jnp.float32)
            return 0

        lax.fori_loop(0, (CH * d) // L, zfill, 0)

        row0 = sid * RT
        for kk in range(WB):
            pltpu.sync_copy(rows0, acc.at[pl.ds(row0 + kk * CH, CH)])

        pltpu.make_async_copy(src_hbm.at[wid, pl.ds(0, GCH)], sbufa, semi).wait()
        pltpu.make_async_copy(dst_hbm.at[wid, pl.ds(0, GCH)], dbufa, semi).wait()
        pltpu.async_copy(h_hbm.at[sbufa.at[0]], rows0, semg0)
        plsc.subcore_barrier()

        bufs = [(sbufa, dbufa), (sbufb, dbufb)]
        for g in range(G):
            sb, db = bufs[g % 2]
            if g < G - 1:
                sbn, dbn = bufs[(g + 1) % 2]
                pltpu.async_copy(
                    src_hbm.at[wid, pl.ds((g + 1) * GCH, GCH)], sbn, semi)
                pltpu.async_copy(
                    dst_hbm.at[wid, pl.ds((g + 1) * GCH, GCH)], dbn, semi)

            def stepj(j, rows, semg, sems, rowsn, semgn, semsn,
                      sb=sb, db=db, first=(g == 0)):
                pltpu.make_async_copy(h_hbm.at[pl.ds(0, CH)], rows, semg).wait()
                if first:
                    @pl.when(j >= 1)
                    def _():
                        drain(rowsn, semsn)
                else:
                    drain(rowsn, semsn)
                pltpu.async_copy(h_hbm.at[sb.at[j + 1]], rowsn, semgn)
                pltpu.async_copy(rows, acc.at[db.at[j]], sems, add=True)

            def inner(j, _):
                @pl.when(j % 2 == 0)
                def _():
                    stepj(j, rows0, semg0, sems0, rows1, semg1, sems1)

                @pl.when(j % 2 == 1)
                def _():
                    stepj(j, rows1, semg1, sems1, rows0, semg0, sems0)

                return 0

            lax.fori_loop(0, GCH - 1, inner, 0)

            pltpu.make_async_copy(h_hbm.at[pl.ds(0, CH)], rows1, semg1).wait()
            if g < G - 1:
                drain(rows0, sems0)
                pltpu.make_async_copy(
                    src_hbm.at[wid, pl.ds(0, GCH)], sbn, semi).wait()
                pltpu.make_async_copy(
                    dst_hbm.at[wid, pl.ds(0, GCH)], dbn, semi).wait()
                pltpu.async_copy(h_hbm.at[sbn.at[0]], rows0, semg0)
            pltpu.async_copy(rows1, acc.at[db.at[GCH - 1]], sems1, add=True)

        drain(rows0, sems0)
        drain(rows1, sems1)
        plsc.subcore_barrier()

        for kk in range(WB):
            pltpu.sync_copy(acc.at[pl.ds(row0 + kk * CH, CH)], rows0)
            pltpu.sync_copy(rows0, out_hbm.at[cid, pl.ds(row0 + kk * CH, CH)])

    return k(h, src3, dst3)


def _tc_norms(deg_partials):
    def k(dp_ref, o_ref):
        deg = dp_ref[0] + dp_ref[1]
        o_ref[...] = lax.rsqrt(jnp.maximum(deg, 1.0))

    return pl.pallas_call(
        k,
        out_shape=jax.ShapeDtypeStruct((2, NPAD), jnp.float32),
    )(deg_partials)


def _tc_pre_matmul(x, ns, w):
    R = 1000

    def k(x_ref, ns_ref, w_ref, o_ref):
        o_ref[...] = jnp.dot(x_ref[...] * ns_ref[...], w_ref[...],
                             preferred_element_type=jnp.float32)

    d_in, d_out = w.shape
    return pl.pallas_call(
        k,
        grid=(N // R,),
        in_specs=[
            pl.BlockSpec((R, d_in), lambda i: (i, 0)),
            pl.BlockSpec((R, 1), lambda i: (i, 0)),
            pl.BlockSpec((d_in, d_out), lambda i: (0, 0)),
        ],
        out_specs=pl.BlockSpec((R, d_out), lambda i: (i, 0)),
        out_shape=jax.ShapeDtypeStruct((N, d_out), jnp.float32),
    )(x, ns, w)


def _tc_mid(partials, nd, ns, b, w):
    R = 1000

    def k(p_ref, nd_ref, ns_ref, b_ref, w_ref, o_ref):
        t = (p_ref[0] + p_ref[1]) * nd_ref[...] + b_ref[...]
        t = jnp.maximum(t, 0.0) * ns_ref[...]
        o_ref[...] = jnp.dot(t, w_ref[...], preferred_element_type=jnp.float32)

    d_in, d_out = w.shape
    return pl.pallas_call(
        k,
        grid=(N // R,),
        in_specs=[
            pl.BlockSpec((NC, R, d_in), lambda i: (0, i, 0)),
            pl.BlockSpec((R, 1), lambda i: (i, 0)),
            pl.BlockSpec((R, 1), lambda i: (i, 0)),
            pl.BlockSpec((1, d_in), lambda i: (0, 0)),
            pl.BlockSpec((d_in, d_out), lambda i: (0, 0)),
        ],
        out_specs=pl.BlockSpec((R, d_out), lambda i: (i, 0)),
        out_shape=jax.ShapeDtypeStruct((N, d_out), jnp.float32),
    )(partials, nd, ns, b, w)


def _tc_elem(partials, nd, ns, b):
    R = 1000

    def k(p_ref, nd_ref, ns_ref, b_ref, o_ref):
        t = (p_ref[0] + p_ref[1]) * nd_ref[...] + b_ref[...]
        o_ref[...] = jnp.maximum(t, 0.0) * ns_ref[...]

    return pl.pallas_call(
        k,
        grid=(N // R,),
        in_specs=[
            pl.BlockSpec((NC, R, D_H), lambda i: (0, i, 0)),
            pl.BlockSpec((R, 1), lambda i: (i, 0)),
            pl.BlockSpec((R, 1), lambda i: (i, 0)),
            pl.BlockSpec((1, D_H), lambda i: (0, 0)),
        ],
        out_specs=pl.BlockSpec((R, D_H), lambda i: (i, 0)),
        out_shape=jax.ShapeDtypeStruct((N, D_H), jnp.float32),
    )(partials, nd, ns, b)


def _tc_final_matmul(partials, nd, b, w):
    R = 1000

    def k(p_ref, nd_ref, b_ref, w_ref, o_ref):
        t = (p_ref[0] + p_ref[1]) * nd_ref[...]
        o_ref[...] = jnp.dot(t, w_ref[...],
                             preferred_element_type=jnp.float32) + b_ref[...]

    d_in, d_out = w.shape
    return pl.pallas_call(
        k,
        grid=(N // R,),
        in_specs=[
            pl.BlockSpec((NC, R, d_in), lambda i: (0, i, 0)),
            pl.BlockSpec((R, 1), lambda i: (i, 0)),
            pl.BlockSpec((1, d_out), lambda i: (0, 0)),
            pl.BlockSpec((d_in, d_out), lambda i: (0, 0)),
        ],
        out_specs=pl.BlockSpec((R, d_out), lambda i: (i, 0)),
        out_shape=jax.ShapeDtypeStruct((N, d_out), jnp.float32),
    )(partials, nd, b, w)


def kernel(x, edge_index, W1, b1, W2, b2, W3, b3):
    src = edge_index[0]
    dst = edge_index[1]

    deg_partials = _sc_degrees(src, dst)
    norms = _tc_norms(deg_partials)
    ns = norms[0, :N, None]
    nd = norms[1, :N, None]

    EPAD = NW * 80 * 128
    src3 = jnp.concatenate(
        [src, jnp.zeros((EPAD - E,), jnp.int32)]).reshape(NW, 80, 128)
    dst3 = jnp.concatenate(
        [dst, jnp.full((EPAD - E,), N, jnp.int32)]).reshape(NW, 80, 128)

    h = _tc_pre_matmul(x, ns, W1)
    p = _sc_spmm(h, src3, dst3, D_H)[:, :N]
    h = _tc_mid(p, nd, ns, b1[None, :], W2)
    p = _sc_spmm(h, src3, dst3, D_H)[:, :N]
    h = _tc_elem(p, nd, ns, b2[None, :])
    p = _sc_spmm(h, src3, dst3, D_H)[:, :N]
    return _tc_final_matmul(p, nd, b3[None, :], W3)


# speedup vs baseline: 2.6877x; 2.6877x over previous
import functools

import jax
import jax.numpy as jnp
from jax import lax
from jax.experimental import pallas as pl
from jax.experimental.pallas import tpu as pltpu
from jax.experimental.pallas import tpu_sc as plsc

N = 10000
E = 320000
D_IN = 128
D_H = 128
D_OUT = 40

NC = 2
NS = 16
NW = NC * NS
L = 16

NPAD = 10240
E_PER_W = E // NW


def _sc_degrees(src, dst):
    CH = 2000
    NT = NPAD // NS

    @functools.partial(
        pl.kernel,
        out_type=jax.ShapeDtypeStruct((NC, 2, NPAD), jnp.float32),
        mesh=plsc.VectorSubcoreMesh(core_axis_name="c", subcore_axis_name="s"),
        scratch_types=[
            pltpu.VMEM((CH,), jnp.int32),
            pltpu.VMEM((CH,), jnp.float32),
            pltpu.VMEM((NT,), jnp.float32),
            pltpu.VMEM_SHARED((NPAD,), jnp.float32),
            pltpu.VMEM_SHARED((NPAD,), jnp.float32),
        ],
    )
    def k(src_hbm, dst_hbm, out_hbm, idxv, onesv, tmpv, acc_s, acc_d):
        cid = lax.axis_index("c")
        sid = lax.axis_index("s")
        wid = cid * NS + sid

        def fill(i, _):
            onesv[pl.ds(i * L, L)] = jnp.full((L,), 1.0, jnp.float32)
            tmpv[pl.ds((i % (NT // L)) * L, L)] = jnp.zeros((L,), jnp.float32)
            return 0

        lax.fori_loop(0, CH // L, fill, 0)

        pltpu.sync_copy(tmpv, acc_s.at[pl.ds(sid * NT, NT)])
        pltpu.sync_copy(tmpv, acc_d.at[pl.ds(sid * NT, NT)])
        plsc.subcore_barrier()

        ebase = wid * E_PER_W

        def chunk(c, _):
            base = ebase + c * CH
            pltpu.sync_copy(src_hbm.at[pl.ds(base, CH)], idxv)
            pltpu.sync_copy(onesv, acc_s.at[idxv], add=True)
            pltpu.sync_copy(dst_hbm.at[pl.ds(base, CH)], idxv)
            pltpu.sync_copy(onesv, acc_d.at[idxv], add=True)
            return 0

        lax.fori_loop(0, E_PER_W // CH, chunk, 0)
        plsc.subcore_barrier()

        pltpu.sync_copy(acc_s.at[pl.ds(sid * NT, NT)], tmpv)
        pltpu.sync_copy(tmpv, out_hbm.at[cid, 0, pl.ds(sid * NT, NT)])
        pltpu.sync_copy(acc_d.at[pl.ds(sid * NT, NT)], tmpv)
        pltpu.sync_copy(tmpv, out_hbm.at[cid, 1, pl.ds(sid * NT, NT)])

    return k(src, dst)


def _sc_spmm(h, src3, dst3, d):
    CH = 128
    RT = NPAD // NS
    GCH = 16
    G = 5
    WB = RT // CH

    @functools.partial(
        pl.kernel,
        out_type=jax.ShapeDtypeStruct((NC, NPAD, d), jnp.float32),
        mesh=plsc.VectorSubcoreMesh(core_axis_name="c", subcore_axis_name="s"),
        scratch_types=[
            pltpu.VMEM((GCH, CH), jnp.int32),
            pltpu.VMEM((GCH, CH), jnp.int32),
            pltpu.VMEM((GCH, CH), jnp.int32),
            pltpu.VMEM((GCH, CH), jnp.int32),
            pltpu.VMEM((CH, d), jnp.float32),
            pltpu.VMEM((CH, d), jnp.float32),
            pltpu.VMEM_SHARED((NPAD, d), jnp.float32),
            pltpu.SemaphoreType.DMA,
            pltpu.SemaphoreType.DMA,
            pltpu.SemaphoreType.DMA,
            pltpu.SemaphoreType.DMA,
            pltpu.SemaphoreType.DMA,
        ],
    )
    def k(h_hbm, src_hbm, dst_hbm, out_hbm,
          sbufa, dbufa, sbufb, dbufb, rows0, rows1, acc,
          semi, semg0, semg1, sems0, sems1):
        cid = lax.axis_index("c")
        sid = lax.axis_index("s")
        wid = cid * NS + sid

        def drain(rows, sems):
            pltpu.make_async_copy(rows, acc.at[pl.ds(0, CH)], sems).wait()

        pltpu.async_copy(src_hbm.at[wid, pl.ds(0, GCH)], sbufa, semi)
        pltpu.async_copy(dst_hbm.at[wid, pl.ds(0, GCH)], dbufa, semi)

        def zfill(i, _):
            r = i // (d // L)
            c = i % (d // L)
            rows0[r, pl.ds(c * L, L)] = jnp.zeros((L,), jnp.float32)
            return 0

        lax.fori_loop(0, (CH * d) // L, zfill, 0)

        row0 = sid * RT
        for kk in range(WB):
            pltpu.sync_copy(rows0, acc.at[pl.ds(row0 + kk * CH, CH)])

        pltpu.make_async_copy(src_hbm.at[wid, pl.ds(0, GCH)], sbufa, semi).wait()
        pltpu.make_async_copy(dst_hbm.at[wid, pl.ds(0, GCH)], dbufa, semi).wait()
        pltpu.async_copy(h_hbm.at[sbufa.at[0]], rows0, semg0)
        plsc.subcore_barrier()

        bufs = [(sbufa, dbufa), (sbufb, dbufb)]
        for g in range(G):
            sb, db = bufs[g % 2]
            if g < G - 1:
                sbn, dbn = bufs[(g + 1) % 2]
                pltpu.async_copy(
                    src_hbm.at[wid, pl.ds((g + 1) * GCH, GCH)], sbn, semi)
                pltpu.async_copy(
                    dst_hbm.at[wid, pl.ds((g + 1) * GCH, GCH)], dbn, semi)

            def stepj(j, rows, semg, sems, rowsn, semgn, semsn,
                      sb=sb, db=db, first=(g == 0)):
                pltpu.make_async_copy(h_hbm.at[pl.ds(0, CH)], rows, semg).wait()
                if first:
                    @pl.when(j >= 1)
                    def _():
                        drain(rowsn, semsn)
                else:
                    drain(rowsn, semsn)
                pltpu.async_copy(h_hbm.at[sb.at[j + 1]], rowsn, semgn)
                pltpu.async_copy(rows, acc.at[db.at[j]], sems, add=True)

            def inner(j, _):
                @pl.when(j % 2 == 0)
                def _():
                    stepj(j, rows0, semg0, sems0, rows1, semg1, sems1)

                @pl.when(j % 2 == 1)
                def _():
                    stepj(j, rows1, semg1, sems1, rows0, semg0, sems0)

                return 0

            lax.fori_loop(0, GCH - 1, inner, 0)

            pltpu.make_async_copy(h_hbm.at[pl.ds(0, CH)], rows1, semg1).wait()
            if g < G - 1:
                drain(rows0, sems0)
                pltpu.make_async_copy(
                    src_hbm.at[wid, pl.ds(0, GCH)], sbn, semi).wait()
                pltpu.make_async_copy(
                    dst_hbm.at[wid, pl.ds(0, GCH)], dbn, semi).wait()
                pltpu.async_copy(h_hbm.at[sbn.at[0]], rows0, semg0)
            pltpu.async_copy(rows1, acc.at[db.at[GCH - 1]], sems1, add=True)

        drain(rows0, sems0)
        drain(rows1, sems1)
        plsc.subcore_barrier()

        for kk in range(WB):
            pltpu.sync_copy(acc.at[pl.ds(row0 + kk * CH, CH)], rows0)
            pltpu.sync_copy(rows0, out_hbm.at[cid, pl.ds(row0 + kk * CH, CH)])

    return k(h, src3, dst3)


def _tc_norms(deg_partials):
    def k(dp_ref, o_ref):
        deg = dp_ref[0] + dp_ref[1]
        o_ref[...] = lax.rsqrt(jnp.maximum(deg, 1.0))

    return pl.pallas_call(
        k,
        out_shape=jax.ShapeDtypeStruct((2, NPAD), jnp.float32),
    )(deg_partials)


def _tc_pre_matmul(x, ns, w):
    R = 1000

    def k(x_ref, ns_ref, w_ref, o_ref):
        o_ref[...] = jnp.dot(x_ref[...] * ns_ref[...], w_ref[...],
                             preferred_element_type=jnp.float32)

    d_in, d_out = w.shape
    return pl.pallas_call(
        k,
        grid=(N // R,),
        in_specs=[
            pl.BlockSpec((R, d_in), lambda i: (i, 0)),
            pl.BlockSpec((R, 1), lambda i: (i, 0)),
            pl.BlockSpec((d_in, d_out), lambda i: (0, 0)),
        ],
        out_specs=pl.BlockSpec((R, d_out), lambda i: (i, 0)),
        out_shape=jax.ShapeDtypeStruct((N, d_out), jnp.float32),
    )(x, ns, w)


def _tc_mid(partials, nd, ns, b, w):
    R = 1000

    def k(p_ref, nd_ref, ns_ref, b_ref, w_ref, o_ref):
        t = (p_ref[0] + p_ref[1]) * nd_ref[...] + b_ref[...]
        t = jnp.maximum(t, 0.0) * ns_ref[...]
        o_ref[...] = jnp.dot(t, w_ref[...], preferred_element_type=jnp.float32)

    d_in, d_out = w.shape
    return pl.pallas_call(
        k,
        grid=(N // R,),
        in_specs=[
            pl.BlockSpec((NC, R, d_in), lambda i: (0, i, 0)),
            pl.BlockSpec((R, 1), lambda i: (i, 0)),
            pl.BlockSpec((R, 1), lambda i: (i, 0)),
            pl.BlockSpec((1, d_in), lambda i: (0, 0)),
            pl.BlockSpec((d_in, d_out), lambda i: (0, 0)),
        ],
        out_specs=pl.BlockSpec((R, d_out), lambda i: (i, 0)),
        out_shape=jax.ShapeDtypeStruct((N, d_out), jnp.float32),
    )(partials, nd, ns, b, w)


def _tc_elem(partials, nd, ns, b):
    R = 1000

    def k(p_ref, nd_ref, ns_ref, b_ref, o_ref):
        t = (p_ref[0] + p_ref[1]) * nd_ref[...] + b_ref[...]
        o_ref[...] = jnp.maximum(t, 0.0) * ns_ref[...]

    return pl.pallas_call(
        k,
        grid=(N // R,),
        in_specs=[
            pl.BlockSpec((NC, R, D_H), lambda i: (0, i, 0)),
            pl.BlockSpec((R, 1), lambda i: (i, 0)),
            pl.BlockSpec((R, 1), lambda i: (i, 0)),
            pl.BlockSpec((1, D_H), lambda i: (0, 0)),
        ],
        out_specs=pl.BlockSpec((R, D_H), lambda i: (i, 0)),
        out_shape=jax.ShapeDtypeStruct((N, D_H), jnp.float32),
    )(partials, nd, ns, b)


def _tc_final_matmul(partials, nd, b, w):
    R = 1000

    def k(p_ref, nd_ref, b_ref, w_ref, o_ref):
        t = (p_ref[0] + p_ref[1]) * nd_ref[...]
        o_ref[...] = jnp.dot(t, w_ref[...],
                             preferred_element_type=jnp.float32) + b_ref[...]

    d_in, d_out = w.shape
    return pl.pallas_call(
        k,
        grid=(N // R,),
        in_specs=[
            pl.BlockSpec((NC, R, d_in), lambda i: (0, i, 0)),
            pl.BlockSpec((R, 1), lambda i: (i, 0)),
            pl.BlockSpec((1, d_out), lambda i: (0, 0)),
            pl.BlockSpec((d_in, d_out), lambda i: (0, 0)),
        ],
        out_specs=pl.BlockSpec((R, d_out), lambda i: (i, 0)),
        out_shape=jax.ShapeDtypeStruct((N, d_out), jnp.float32),
    )(partials, nd, b, w)


def kernel(x, edge_index, W1, b1, W2, b2, W3, b3):
    src = edge_index[0]
    dst = edge_index[1]

    deg_partials = _sc_degrees(src, dst)
    norms = _tc_norms(deg_partials)
    ns = norms[0, :N, None]
    nd = norms[1, :N, None]

    EPAD = NW * 80 * 128
    pad_ids = jnp.arange(EPAD - E, dtype=jnp.int32)
    src3 = jnp.concatenate(
        [src, pad_ids % N]).reshape(NW, 80, 128)
    dst3 = jnp.concatenate(
        [dst, N + pad_ids % (NPAD - N)]).reshape(NW, 80, 128)

    h = _tc_pre_matmul(x, ns, W1)
    p = _sc_spmm(h, src3, dst3, D_H)[:, :N]
    h = _tc_mid(p, nd, ns, b1[None, :], W2)
    p = _sc_spmm(h, src3, dst3, D_H)[:, :N]
    h = _tc_elem(p, nd, ns, b2[None, :])
    p = _sc_spmm(h, src3, dst3, D_H)[:, :N]
    return _tc_final_matmul(p, nd, b3[None, :], W3)


# speedup vs baseline: 3.3440x; 1.2442x over previous
import functools

import jax
import jax.numpy as jnp
from jax import lax
from jax.experimental import pallas as pl
from jax.experimental.pallas import tpu as pltpu
from jax.experimental.pallas import tpu_sc as plsc

N = 10000
E = 320000
D_IN = 128
D_H = 128
D_OUT = 40

NC = 2
NS = 16
NW = NC * NS
L = 16

NPAD = 10240
E_PER_W = E // NW


def _sc_degrees(src, dst):
    CH = 2000
    NT = NPAD // NS

    @functools.partial(
        pl.kernel,
        out_type=jax.ShapeDtypeStruct((NC, 2, NPAD), jnp.float32),
        mesh=plsc.VectorSubcoreMesh(core_axis_name="c", subcore_axis_name="s"),
        scratch_types=[
            pltpu.VMEM((CH,), jnp.int32),
            pltpu.VMEM((CH,), jnp.float32),
            pltpu.VMEM((NT,), jnp.float32),
            pltpu.VMEM_SHARED((NPAD,), jnp.float32),
            pltpu.VMEM_SHARED((NPAD,), jnp.float32),
        ],
    )
    def k(src_hbm, dst_hbm, out_hbm, idxv, onesv, tmpv, acc_s, acc_d):
        cid = lax.axis_index("c")
        sid = lax.axis_index("s")
        wid = cid * NS + sid

        def fill(i, _):
            onesv[pl.ds(i * L, L)] = jnp.full((L,), 1.0, jnp.float32)
            tmpv[pl.ds((i % (NT // L)) * L, L)] = jnp.zeros((L,), jnp.float32)
            return 0

        lax.fori_loop(0, CH // L, fill, 0)

        pltpu.sync_copy(tmpv, acc_s.at[pl.ds(sid * NT, NT)])
        pltpu.sync_copy(tmpv, acc_d.at[pl.ds(sid * NT, NT)])
        plsc.subcore_barrier()

        ebase = wid * E_PER_W

        def chunk(c, _):
            base = ebase + c * CH
            pltpu.sync_copy(src_hbm.at[pl.ds(base, CH)], idxv)
            pltpu.sync_copy(onesv, acc_s.at[idxv], add=True)
            pltpu.sync_copy(dst_hbm.at[pl.ds(base, CH)], idxv)
            pltpu.sync_copy(onesv, acc_d.at[idxv], add=True)
            return 0

        lax.fori_loop(0, E_PER_W // CH, chunk, 0)
        plsc.subcore_barrier()

        pltpu.sync_copy(acc_s.at[pl.ds(sid * NT, NT)], tmpv)
        pltpu.sync_copy(tmpv, out_hbm.at[cid, 0, pl.ds(sid * NT, NT)])
        pltpu.sync_copy(acc_d.at[pl.ds(sid * NT, NT)], tmpv)
        pltpu.sync_copy(tmpv, out_hbm.at[cid, 1, pl.ds(sid * NT, NT)])

    return k(src, dst)


def _sc_spmm(h, src3, dst3, d):
    CH = 64
    DEP = 4
    RT = NPAD // NS
    GCH = 16
    G = 10
    WB = RT // CH

    @functools.partial(
        pl.kernel,
        out_type=jax.ShapeDtypeStruct((NC, NPAD, d), jnp.float32),
        mesh=plsc.VectorSubcoreMesh(core_axis_name="c", subcore_axis_name="s"),
        scratch_types=[
            pltpu.VMEM((GCH, CH), jnp.int32),
            pltpu.VMEM((GCH, CH), jnp.int32),
            pltpu.VMEM((GCH, CH), jnp.int32),
            pltpu.VMEM((GCH, CH), jnp.int32),
            pltpu.VMEM((CH, d), jnp.float32),
            pltpu.VMEM((CH, d), jnp.float32),
            pltpu.VMEM((CH, d), jnp.float32),
            pltpu.VMEM((CH, d), jnp.float32),
            pltpu.VMEM_SHARED((NPAD, d), jnp.float32),
            pltpu.SemaphoreType.DMA,
            pltpu.SemaphoreType.DMA,
            pltpu.SemaphoreType.DMA,
            pltpu.SemaphoreType.DMA,
            pltpu.SemaphoreType.DMA,
            pltpu.SemaphoreType.DMA,
            pltpu.SemaphoreType.DMA,
            pltpu.SemaphoreType.DMA,
            pltpu.SemaphoreType.DMA,
        ],
    )
    def k(h_hbm, src_hbm, dst_hbm, out_hbm,
          sbufa, dbufa, sbufb, dbufb, r0, r1, r2, r3, acc,
          semi, sg0, sg1, sg2, sg3, ss0, ss1, ss2, ss3):
        cid = lax.axis_index("c")
        sid = lax.axis_index("s")
        wid = cid * NS + sid
        rows = [r0, r1, r2, r3]
        sg = [sg0, sg1, sg2, sg3]
        ss = [ss0, ss1, ss2, ss3]

        def gwait(b):
            pltpu.make_async_copy(h_hbm.at[pl.ds(0, CH)], rows[b], sg[b]).wait()

        def drain(b):
            pltpu.make_async_copy(rows[b], acc.at[pl.ds(0, CH)], ss[b]).wait()

        def gissue(idxrow, b):
            pltpu.async_copy(h_hbm.at[idxrow], rows[b], sg[b])

        def sissue(j, db, b):
            pltpu.async_copy(rows[b], acc.at[db.at[j]], ss[b], add=True)

        pltpu.async_copy(src_hbm.at[wid, pl.ds(0, GCH)], sbufa, semi)
        pltpu.async_copy(dst_hbm.at[wid, pl.ds(0, GCH)], dbufa, semi)

        def zfill(i, _):
            r = i // (d // L)
            c = i % (d // L)
            r0[r, pl.ds(c * L, L)] = jnp.zeros((L,), jnp.float32)
            return 0

        lax.fori_loop(0, (CH * d) // L, zfill, 0)

        row0 = sid * RT
        for kk in range(WB):
            pltpu.sync_copy(r0, acc.at[pl.ds(row0 + kk * CH, CH)])

        pltpu.make_async_copy(src_hbm.at[wid, pl.ds(0, GCH)], sbufa, semi).wait()
        pltpu.make_async_copy(dst_hbm.at[wid, pl.ds(0, GCH)], dbufa, semi).wait()
        gissue(sbufa.at[0], 0)
        gissue(sbufa.at[1], 1)
        gissue(sbufa.at[2], 2)
        plsc.subcore_barrier()

        bufs = [(sbufa, dbufa), (sbufb, dbufb)]
        for g in range(G):
            sb, db = bufs[g % 2]
            if g < G - 1:
                sbn, dbn = bufs[(g + 1) % 2]
                pltpu.async_copy(
                    src_hbm.at[wid, pl.ds((g + 1) * GCH, GCH)], sbn, semi)
                pltpu.async_copy(
                    dst_hbm.at[wid, pl.ds((g + 1) * GCH, GCH)], dbn, semi)

            def stepj(j, b, sb=sb, db=db, first=(g == 0)):
                gwait(b)
                if first:
                    @pl.when(j >= 1)
                    def _():
                        drain((b + 3) % DEP)
                else:
                    drain((b + 3) % DEP)
                pltpu.async_copy(
                    h_hbm.at[sb.at[j + 3]], rows[(b + 3) % DEP],
                    sg[(b + 3) % DEP])
                sissue(j, db, b)

            def inner(j, _):
                for b in range(DEP):
                    @pl.when(j % DEP == b)
                    def _(j=j, b=b):
                        stepj(j, b)
                return 0

            lax.fori_loop(0, GCH - 3, inner, 0)

            for j in range(GCH - 3, GCH):
                b = j % DEP
                gwait(b)
                drain((b + 3) % DEP)
                if g < G - 1:
                    if j == GCH - 3:
                        pltpu.make_async_copy(
                            src_hbm.at[wid, pl.ds(0, GCH)], sbn, semi).wait()
                        pltpu.make_async_copy(
                            dst_hbm.at[wid, pl.ds(0, GCH)], dbn, semi).wait()
                    gissue(sbn.at[j - (GCH - 3)], (b + 3) % DEP)
                sissue(j, db, b)

        drain((GCH - 1) % DEP)
        plsc.subcore_barrier()

        for kk in range(WB):
            pltpu.sync_copy(acc.at[pl.ds(row0 + kk * CH, CH)], r0)
            pltpu.sync_copy(r0, out_hbm.at[cid, pl.ds(row0 + kk * CH, CH)])

    return k(h, src3, dst3)


def _tc_norms(deg_partials):
    def k(dp_ref, o_ref):
        deg = dp_ref[0] + dp_ref[1]
        o_ref[...] = lax.rsqrt(jnp.maximum(deg, 1.0))

    return pl.pallas_call(
        k,
        out_shape=jax.ShapeDtypeStruct((2, NPAD), jnp.float32),
    )(deg_partials)


def _tc_pre_matmul(x, ns, w):
    R = 1000

    def k(x_ref, ns_ref, w_ref, o_ref):
        o_ref[...] = jnp.dot(x_ref[...] * ns_ref[...], w_ref[...],
                             preferred_element_type=jnp.float32)

    d_in, d_out = w.shape
    return pl.pallas_call(
        k,
        grid=(N // R,),
        in_specs=[
            pl.BlockSpec((R, d_in), lambda i: (i, 0)),
            pl.BlockSpec((R, 1), lambda i: (i, 0)),
            pl.BlockSpec((d_in, d_out), lambda i: (0, 0)),
        ],
        out_specs=pl.BlockSpec((R, d_out), lambda i: (i, 0)),
        out_shape=jax.ShapeDtypeStruct((N, d_out), jnp.float32),
    )(x, ns, w)


def _tc_mid(partials, nd, ns, b, w):
    R = 1000

    def k(p_ref, nd_ref, ns_ref, b_ref, w_ref, o_ref):
        t = (p_ref[0] + p_ref[1]) * nd_ref[...] + b_ref[...]
        t = jnp.maximum(t, 0.0) * ns_ref[...]
        o_ref[...] = jnp.dot(t, w_ref[...], preferred_element_type=jnp.float32)

    d_in, d_out = w.shape
    return pl.pallas_call(
        k,
        grid=(N // R,),
        in_specs=[
            pl.BlockSpec((NC, R, d_in), lambda i: (0, i, 0)),
            pl.BlockSpec((R, 1), lambda i: (i, 0)),
            pl.BlockSpec((R, 1), lambda i: (i, 0)),
            pl.BlockSpec((1, d_in), lambda i: (0, 0)),
            pl.BlockSpec((d_in, d_out), lambda i: (0, 0)),
        ],
        out_specs=pl.BlockSpec((R, d_out), lambda i: (i, 0)),
        out_shape=jax.ShapeDtypeStruct((N, d_out), jnp.float32),
    )(partials, nd, ns, b, w)


def _tc_elem(partials, nd, ns, b):
    R = 1000

    def k(p_ref, nd_ref, ns_ref, b_ref, o_ref):
        t = (p_ref[0] + p_ref[1]) * nd_ref[...] + b_ref[...]
        o_ref[...] = jnp.maximum(t, 0.0) * ns_ref[...]

    return pl.pallas_call(
        k,
        grid=(N // R,),
        in_specs=[
            pl.BlockSpec((NC, R, D_H), lambda i: (0, i, 0)),
            pl.BlockSpec((R, 1), lambda i: (i, 0)),
            pl.BlockSpec((R, 1), lambda i: (i, 0)),
            pl.BlockSpec((1, D_H), lambda i: (0, 0)),
        ],
        out_specs=pl.BlockSpec((R, D_H), lambda i: (i, 0)),
        out_shape=jax.ShapeDtypeStruct((N, D_H), jnp.float32),
    )(partials, nd, ns, b)


def _tc_final_matmul(partials, nd, b, w):
    R = 1000

    def k(p_ref, nd_ref, b_ref, w_ref, o_ref):
        t = (p_ref[0] + p_ref[1]) * nd_ref[...]
        o_ref[...] = jnp.dot(t, w_ref[...],
                             preferred_element_type=jnp.float32) + b_ref[...]

    d_in, d_out = w.shape
    return pl.pallas_call(
        k,
        grid=(N // R,),
        in_specs=[
            pl.BlockSpec((NC, R, d_in), lambda i: (0, i, 0)),
            pl.BlockSpec((R, 1), lambda i: (i, 0)),
            pl.BlockSpec((1, d_out), lambda i: (0, 0)),
            pl.BlockSpec((d_in, d_out), lambda i: (0, 0)),
        ],
        out_specs=pl.BlockSpec((R, d_out), lambda i: (i, 0)),
        out_shape=jax.ShapeDtypeStruct((N, d_out), jnp.float32),
    )(partials, nd, b, w)


def kernel(x, edge_index, W1, b1, W2, b2, W3, b3):
    src = edge_index[0]
    dst = edge_index[1]

    deg_partials = _sc_degrees(src, dst)
    norms = _tc_norms(deg_partials)
    ns = norms[0, :N, None]
    nd = norms[1, :N, None]

    EPAD = NW * 80 * 128
    pad_ids = jnp.arange(EPAD - E, dtype=jnp.int32)
    src3 = jnp.concatenate(
        [src, pad_ids % N]).reshape(NW, 160, 64)
    dst3 = jnp.concatenate(
        [dst, N + pad_ids % (NPAD - N)]).reshape(NW, 160, 64)

    h = _tc_pre_matmul(x, ns, W1)
    p = _sc_spmm(h, src3, dst3, D_H)[:, :N]
    h = _tc_mid(p, nd, ns, b1[None, :], W2)
    p = _sc_spmm(h, src3, dst3, D_H)[:, :N]
    h = _tc_elem(p, nd, ns, b2[None, :])
    p = _sc_spmm(h, src3, dst3, D_H)[:, :N]
    return _tc_final_matmul(p, nd, b3[None, :], W3)


# speedup vs baseline: 3.5223x; 1.0533x over previous
import functools

import jax
import jax.numpy as jnp
from jax import lax
from jax.experimental import pallas as pl
from jax.experimental.pallas import tpu as pltpu
from jax.experimental.pallas import tpu_sc as plsc

N = 10000
E = 320000
D_IN = 128
D_H = 128
D_OUT = 40

NC = 2
NS = 16
NW = NC * NS
L = 16

NPAD = 10240
E_PER_W = E // NW


def _sc_degrees(src, dst):
    CH = 2000
    NT = NPAD // NS

    @functools.partial(
        pl.kernel,
        out_type=jax.ShapeDtypeStruct((NC, 2, NPAD), jnp.float32),
        mesh=plsc.VectorSubcoreMesh(core_axis_name="c", subcore_axis_name="s"),
        scratch_types=[
            pltpu.VMEM((CH,), jnp.int32),
            pltpu.VMEM((CH,), jnp.float32),
            pltpu.VMEM((NT,), jnp.float32),
            pltpu.VMEM_SHARED((NPAD,), jnp.float32),
            pltpu.VMEM_SHARED((NPAD,), jnp.float32),
        ],
    )
    def k(src_hbm, dst_hbm, out_hbm, idxv, onesv, tmpv, acc_s, acc_d):
        cid = lax.axis_index("c")
        sid = lax.axis_index("s")
        wid = cid * NS + sid

        def fill(i, _):
            onesv[pl.ds(i * L, L)] = jnp.full((L,), 1.0, jnp.float32)
            tmpv[pl.ds((i % (NT // L)) * L, L)] = jnp.zeros((L,), jnp.float32)
            return 0

        lax.fori_loop(0, CH // L, fill, 0)

        pltpu.sync_copy(tmpv, acc_s.at[pl.ds(sid * NT, NT)])
        pltpu.sync_copy(tmpv, acc_d.at[pl.ds(sid * NT, NT)])
        plsc.subcore_barrier()

        ebase = wid * E_PER_W

        def chunk(c, _):
            base = ebase + c * CH
            pltpu.sync_copy(src_hbm.at[pl.ds(base, CH)], idxv)
            pltpu.sync_copy(onesv, acc_s.at[idxv], add=True)
            pltpu.sync_copy(dst_hbm.at[pl.ds(base, CH)], idxv)
            pltpu.sync_copy(onesv, acc_d.at[idxv], add=True)
            return 0

        lax.fori_loop(0, E_PER_W // CH, chunk, 0)
        plsc.subcore_barrier()

        pltpu.sync_copy(acc_s.at[pl.ds(sid * NT, NT)], tmpv)
        pltpu.sync_copy(tmpv, out_hbm.at[cid, 0, pl.ds(sid * NT, NT)])
        pltpu.sync_copy(acc_d.at[pl.ds(sid * NT, NT)], tmpv)
        pltpu.sync_copy(tmpv, out_hbm.at[cid, 1, pl.ds(sid * NT, NT)])

    return k(src, dst)


def _sc_spmm(h, src3, dst3, d):
    CH = 64
    DEP = 4
    RT = NPAD // NS
    GCH = 16
    G = 10
    WB = RT // CH

    @functools.partial(
        pl.kernel,
        out_type=jax.ShapeDtypeStruct((NC, NPAD, d), jnp.float32),
        mesh=plsc.VectorSubcoreMesh(core_axis_name="c", subcore_axis_name="s"),
        scratch_types=[
            pltpu.VMEM((GCH, CH), jnp.int32),
            pltpu.VMEM((GCH, CH), jnp.int32),
            pltpu.VMEM((GCH, CH), jnp.int32),
            pltpu.VMEM((GCH, CH), jnp.int32),
            pltpu.VMEM((CH, d), jnp.float32),
            pltpu.VMEM((CH, d), jnp.float32),
            pltpu.VMEM((CH, d), jnp.float32),
            pltpu.VMEM((CH, d), jnp.float32),
            pltpu.VMEM_SHARED((NPAD, d), jnp.float32),
            pltpu.SemaphoreType.DMA,
            pltpu.SemaphoreType.DMA,
            pltpu.SemaphoreType.DMA,
            pltpu.SemaphoreType.DMA,
            pltpu.SemaphoreType.DMA,
            pltpu.SemaphoreType.DMA,
            pltpu.SemaphoreType.DMA,
            pltpu.SemaphoreType.DMA,
            pltpu.SemaphoreType.DMA,
        ],
    )
    def k(h_hbm, src_hbm, dst_hbm, out_hbm,
          sbufa, dbufa, sbufb, dbufb, r0, r1, r2, r3, acc,
          semi, sg0, sg1, sg2, sg3, ss0, ss1, ss2, ss3):
        cid = lax.axis_index("c")
        sid = lax.axis_index("s")
        wid = cid * NS + sid
        rows = [r0, r1, r2, r3]
        sg = [sg0, sg1, sg2, sg3]
        ss = [ss0, ss1, ss2, ss3]

        def gwait(b):
            pltpu.make_async_copy(h_hbm.at[pl.ds(0, CH)], rows[b], sg[b]).wait()

        def drain(b):
            pltpu.make_async_copy(rows[b], acc.at[pl.ds(0, CH)], ss[b]).wait()

        def gissue(idxrow, b):
            pltpu.async_copy(h_hbm.at[idxrow], rows[b], sg[b])

        def sissue(j, db, b):
            pltpu.async_copy(rows[b], acc.at[db.at[j]], ss[b], add=True)

        pltpu.async_copy(src_hbm.at[wid, pl.ds(0, GCH)], sbufa, semi)
        pltpu.async_copy(dst_hbm.at[wid, pl.ds(0, GCH)], dbufa, semi)

        def zfill(i, _):
            r = i // (d // L)
            c = i % (d // L)
            r0[r, pl.ds(c * L, L)] = jnp.zeros((L,), jnp.float32)
            return 0

        lax.fori_loop(0, (CH * d) // L, zfill, 0)

        row0 = sid * RT
        for kk in range(WB):
            pltpu.sync_copy(r0, acc.at[pl.ds(row0 + kk * CH, CH)])

        pltpu.make_async_copy(src_hbm.at[wid, pl.ds(0, GCH)], sbufa, semi).wait()
        pltpu.make_async_copy(dst_hbm.at[wid, pl.ds(0, GCH)], dbufa, semi).wait()
        gissue(sbufa.at[0], 0)
        gissue(sbufa.at[1], 1)
        gissue(sbufa.at[2], 2)
        plsc.subcore_barrier()

        bufs = [(sbufa, dbufa), (sbufb, dbufb)]
        for g in range(G):
            sb, db = bufs[g % 2]
            if g < G - 1:
                sbn, dbn = bufs[(g + 1) % 2]
                pltpu.async_copy(
                    src_hbm.at[wid, pl.ds((g + 1) * GCH, GCH)], sbn, semi)
                pltpu.async_copy(
                    dst_hbm.at[wid, pl.ds((g + 1) * GCH, GCH)], dbn, semi)

            def stepj(j, b, sb=sb, db=db, first=(g == 0)):
                gwait(b)
                if first:
                    @pl.when(j >= 1)
                    def _():
                        drain((b + 3) % DEP)
                else:
                    drain((b + 3) % DEP)
                pltpu.async_copy(
                    h_hbm.at[sb.at[j + 3]], rows[(b + 3) % DEP],
                    sg[(b + 3) % DEP])
                sissue(j, db, b)

            def inner(j, _):
                for b in range(DEP):
                    @pl.when(j % DEP == b)
                    def _(j=j, b=b):
                        stepj(j, b)
                return 0

            lax.fori_loop(0, GCH - 3, inner, 0)

            for j in range(GCH - 3, GCH):
                b = j % DEP
                gwait(b)
                drain((b + 3) % DEP)
                if g < G - 1:
                    if j == GCH - 3:
                        pltpu.make_async_copy(
                            src_hbm.at[wid, pl.ds(0, GCH)], sbn, semi).wait()
                        pltpu.make_async_copy(
                            dst_hbm.at[wid, pl.ds(0, GCH)], dbn, semi).wait()
                    gissue(sbn.at[j - (GCH - 3)], (b + 3) % DEP)
                sissue(j, db, b)

        drain((GCH - 1) % DEP)
        plsc.subcore_barrier()

        for kk in range(WB):
            pltpu.sync_copy(acc.at[pl.ds(row0 + kk * CH, CH)], r0)
            pltpu.sync_copy(r0, out_hbm.at[cid, pl.ds(row0 + kk * CH, CH)])

    return k(h, src3, dst3)


def _tc_norms(deg_partials):
    def k(dp_ref, o_ref):
        deg = dp_ref[0] + dp_ref[1]
        o_ref[...] = lax.rsqrt(jnp.maximum(deg, 1.0))

    return pl.pallas_call(
        k,
        out_shape=jax.ShapeDtypeStruct((2, NPAD), jnp.float32),
    )(deg_partials)


def _tc_pre_matmul(x, ns, w):
    R = 1000

    def k(x_ref, ns_ref, w_ref, o_ref):
        o_ref[...] = jnp.dot(x_ref[...] * ns_ref[...], w_ref[...],
                             preferred_element_type=jnp.float32)

    d_in, d_out = w.shape
    return pl.pallas_call(
        k,
        grid=(N // R,),
        in_specs=[
            pl.BlockSpec((R, d_in), lambda i: (i, 0)),
            pl.BlockSpec((R, 1), lambda i: (i, 0)),
            pl.BlockSpec((d_in, d_out), lambda i: (0, 0)),
        ],
        out_specs=pl.BlockSpec((R, d_out), lambda i: (i, 0)),
        out_shape=jax.ShapeDtypeStruct((N, d_out), jnp.float32),
    )(x, ns, w)


def _tc_mid(partials, nd, ns, b, w):
    R = 1000

    def k(p_ref, nd_ref, ns_ref, b_ref, w_ref, o_ref):
        t = (p_ref[0] + p_ref[1]) * nd_ref[...] + b_ref[...]
        t = jnp.maximum(t, 0.0) * ns_ref[...]
        o_ref[...] = jnp.dot(t, w_ref[...], preferred_element_type=jnp.float32)

    d_in, d_out = w.shape
    return pl.pallas_call(
        k,
        grid=(N // R,),
        in_specs=[
            pl.BlockSpec((NC, R, d_in), lambda i: (0, i, 0)),
            pl.BlockSpec((R, 1), lambda i: (i, 0)),
            pl.BlockSpec((R, 1), lambda i: (i, 0)),
            pl.BlockSpec((1, d_in), lambda i: (0, 0)),
            pl.BlockSpec((d_in, d_out), lambda i: (0, 0)),
        ],
        out_specs=pl.BlockSpec((R, d_out), lambda i: (i, 0)),
        out_shape=jax.ShapeDtypeStruct((N, d_out), jnp.float32),
    )(partials, nd, ns, b, w)


def _tc_elem(partials, nd, ns, b):
    R = 1000

    def k(p_ref, nd_ref, ns_ref, b_ref, o_ref):
        t = (p_ref[0] + p_ref[1]) * nd_ref[...] + b_ref[...]
        o_ref[...] = jnp.maximum(t, 0.0) * ns_ref[...]

    return pl.pallas_call(
        k,
        grid=(N // R,),
        in_specs=[
            pl.BlockSpec((NC, R, D_H), lambda i: (0, i, 0)),
            pl.BlockSpec((R, 1), lambda i: (i, 0)),
            pl.BlockSpec((R, 1), lambda i: (i, 0)),
            pl.BlockSpec((1, D_H), lambda i: (0, 0)),
        ],
        out_specs=pl.BlockSpec((R, D_H), lambda i: (i, 0)),
        out_shape=jax.ShapeDtypeStruct((N, D_H), jnp.float32),
    )(partials, nd, ns, b)


def _tc_final_matmul(partials, nd, b, w):
    R = 1000

    def k(p_ref, nd_ref, b_ref, w_ref, o_ref):
        t = (p_ref[0] + p_ref[1]) * nd_ref[...]
        o_ref[...] = jnp.dot(t, w_ref[...],
                             preferred_element_type=jnp.float32) + b_ref[...]

    d_in, d_out = w.shape
    return pl.pallas_call(
        k,
        grid=(N // R,),
        in_specs=[
            pl.BlockSpec((NC, R, d_in), lambda i: (0, i, 0)),
            pl.BlockSpec((R, 1), lambda i: (i, 0)),
            pl.BlockSpec((1, d_out), lambda i: (0, 0)),
            pl.BlockSpec((d_in, d_out), lambda i: (0, 0)),
        ],
        out_specs=pl.BlockSpec((R, d_out), lambda i: (i, 0)),
        out_shape=jax.ShapeDtypeStruct((N, d_out), jnp.float32),
    )(partials, nd, b, w)


def kernel(x, edge_index, W1, b1, W2, b2, W3, b3):
    src = edge_index[0]
    dst = edge_index[1]

    deg_partials = _sc_degrees(src, dst)
    norms = _tc_norms(deg_partials)
    ns = norms[0, :N, None]
    nd = norms[1, :N, None]

    EPAD = NW * 80 * 128
    pad_ids = jnp.arange(EPAD - E, dtype=jnp.int32)
    src3 = jnp.concatenate(
        [src, pad_ids % N]).reshape(NW, 160, 64)
    dst3 = jnp.concatenate(
        [dst, N + pad_ids % (NPAD - N)]).reshape(NW, 160, 64)

    h = _tc_pre_matmul(x, ns, W1)
    p = _sc_spmm(h, src3, dst3, D_H)
    h = _tc_mid(p, nd, ns, b1[None, :], W2)
    p = _sc_spmm(h, src3, dst3, D_H)
    h = _tc_elem(p, nd, ns, b2[None, :])
    p = _sc_spmm(h, src3, dst3, D_H)
    return _tc_final_matmul(p, nd, b3[None, :], W3)


# speedup vs baseline: 3.5737x; 1.0146x over previous
import functools

import jax
import jax.numpy as jnp
from jax import lax
from jax.experimental import pallas as pl
from jax.experimental.pallas import tpu as pltpu
from jax.experimental.pallas import tpu_sc as plsc

N = 10000
E = 320000
D_IN = 128
D_H = 128
D_OUT = 40

NC = 2
NS = 16
NW = NC * NS
L = 16

NPAD = 10240
SPAD = 10112
E_PER_W = E // NW


def _sc_degrees(src, dst):
    CH = 2000
    NT = NPAD // NS

    @functools.partial(
        pl.kernel,
        out_type=jax.ShapeDtypeStruct((NC, 2, NPAD), jnp.float32),
        mesh=plsc.VectorSubcoreMesh(core_axis_name="c", subcore_axis_name="s"),
        scratch_types=[
            pltpu.VMEM((CH,), jnp.int32),
            pltpu.VMEM((CH,), jnp.float32),
            pltpu.VMEM((NT,), jnp.float32),
            pltpu.VMEM_SHARED((NPAD,), jnp.float32),
            pltpu.VMEM_SHARED((NPAD,), jnp.float32),
        ],
    )
    def k(src_hbm, dst_hbm, out_hbm, idxv, onesv, tmpv, acc_s, acc_d):
        cid = lax.axis_index("c")
        sid = lax.axis_index("s")
        wid = cid * NS + sid

        def fill(i, _):
            onesv[pl.ds(i * L, L)] = jnp.full((L,), 1.0, jnp.float32)
            tmpv[pl.ds((i % (NT // L)) * L, L)] = jnp.zeros((L,), jnp.float32)
            return 0

        lax.fori_loop(0, CH // L, fill, 0)

        pltpu.sync_copy(tmpv, acc_s.at[pl.ds(sid * NT, NT)])
        pltpu.sync_copy(tmpv, acc_d.at[pl.ds(sid * NT, NT)])
        plsc.subcore_barrier()

        ebase = wid * E_PER_W

        def chunk(c, _):
            base = ebase + c * CH
            pltpu.sync_copy(src_hbm.at[pl.ds(base, CH)], idxv)
            pltpu.sync_copy(onesv, acc_s.at[idxv], add=True)
            pltpu.sync_copy(dst_hbm.at[pl.ds(base, CH)], idxv)
            pltpu.sync_copy(onesv, acc_d.at[idxv], add=True)
            return 0

        lax.fori_loop(0, E_PER_W // CH, chunk, 0)
        plsc.subcore_barrier()

        pltpu.sync_copy(acc_s.at[pl.ds(sid * NT, NT)], tmpv)
        pltpu.sync_copy(tmpv, out_hbm.at[cid, 0, pl.ds(sid * NT, NT)])
        pltpu.sync_copy(acc_d.at[pl.ds(sid * NT, NT)], tmpv)
        pltpu.sync_copy(tmpv, out_hbm.at[cid, 1, pl.ds(sid * NT, NT)])

    return k(src, dst)


def _sc_spmm(h, src3, dst3, d):
    CH = 64
    DEP = 5
    LOOK = DEP - 1
    RT = SPAD // NS
    GCH = 16
    G = 10
    WBS = [CH] * (RT // CH) + [RT - (RT // CH) * CH]

    @functools.partial(
        pl.kernel,
        out_type=jax.ShapeDtypeStruct((NC, SPAD, d), jnp.float32),
        mesh=plsc.VectorSubcoreMesh(core_axis_name="c", subcore_axis_name="s"),
        scratch_types=[
            pltpu.VMEM((GCH, CH), jnp.int32),
            pltpu.VMEM((GCH, CH), jnp.int32),
            pltpu.VMEM((GCH, CH), jnp.int32),
            pltpu.VMEM((GCH, CH), jnp.int32),
            pltpu.VMEM((CH, d), jnp.float32),
            pltpu.VMEM((CH, d), jnp.float32),
            pltpu.VMEM((CH, d), jnp.float32),
            pltpu.VMEM((CH, d), jnp.float32),
            pltpu.VMEM((CH, d), jnp.float32),
            pltpu.VMEM_SHARED((SPAD, d), jnp.float32),
            pltpu.SemaphoreType.DMA,
            pltpu.SemaphoreType.DMA,
            pltpu.SemaphoreType.DMA,
            pltpu.SemaphoreType.DMA,
            pltpu.SemaphoreType.DMA,
            pltpu.SemaphoreType.DMA,
            pltpu.SemaphoreType.DMA,
            pltpu.SemaphoreType.DMA,
            pltpu.SemaphoreType.DMA,
            pltpu.SemaphoreType.DMA,
            pltpu.SemaphoreType.DMA,
        ],
    )
    def k(h_hbm, src_hbm, dst_hbm, out_hbm,
          sbufa, dbufa, sbufb, dbufb, r0, r1, r2, r3, r4, acc,
          semi, sg0, sg1, sg2, sg3, sg4, ss0, ss1, ss2, ss3, ss4):
        cid = lax.axis_index("c")
        sid = lax.axis_index("s")
        wid = cid * NS + sid
        rows = [r0, r1, r2, r3, r4]
        sg = [sg0, sg1, sg2, sg3, sg4]
        ss = [ss0, ss1, ss2, ss3, ss4]

        def gwait(b):
            pltpu.make_async_copy(h_hbm.at[pl.ds(0, CH)], rows[b], sg[b]).wait()

        def drain(b):
            pltpu.make_async_copy(rows[b], acc.at[pl.ds(0, CH)], ss[b]).wait()

        def gissue(idxrow, b):
            pltpu.async_copy(h_hbm.at[idxrow], rows[b], sg[b])

        def sissue(j, db, b):
            pltpu.async_copy(rows[b], acc.at[db.at[j]], ss[b], add=True)

        pltpu.async_copy(src_hbm.at[wid, pl.ds(0, GCH)], sbufa, semi)
        pltpu.async_copy(dst_hbm.at[wid, pl.ds(0, GCH)], dbufa, semi)

        def zfill(i, _):
            r = i // (d // L)
            c = i % (d // L)
            r0[r, pl.ds(c * L, L)] = jnp.zeros((L,), jnp.float32)
            return 0

        lax.fori_loop(0, (CH * d) // L, zfill, 0)

        row0 = sid * RT
        ro = 0
        for sz in WBS:
            pltpu.sync_copy(r0.at[pl.ds(0, sz)],
                            acc.at[pl.ds(row0 + ro, sz)])
            ro += sz

        pltpu.make_async_copy(src_hbm.at[wid, pl.ds(0, GCH)], sbufa, semi).wait()
        pltpu.make_async_copy(dst_hbm.at[wid, pl.ds(0, GCH)], dbufa, semi).wait()
        for b in range(LOOK):
            gissue(sbufa.at[b], b)
        plsc.subcore_barrier()

        bufs = [(sbufa, dbufa), (sbufb, dbufb)]
        for g in range(G):
            sb, db = bufs[g % 2]
            off = (g * GCH) % DEP
            if g < G - 1:
                sbn, dbn = bufs[(g + 1) % 2]
                pltpu.async_copy(
                    src_hbm.at[wid, pl.ds((g + 1) * GCH, GCH)], sbn, semi)
                pltpu.async_copy(
                    dst_hbm.at[wid, pl.ds((g + 1) * GCH, GCH)], dbn, semi)

            def stepj(j, b, sb=sb, db=db, first=(g == 0)):
                gwait(b)
                if first:
                    @pl.when(j >= 1)
                    def _():
                        drain((b + LOOK) % DEP)
                else:
                    drain((b + LOOK) % DEP)
                pltpu.async_copy(
                    h_hbm.at[sb.at[j + LOOK]], rows[(b + LOOK) % DEP],
                    sg[(b + LOOK) % DEP])
                sissue(j, db, b)

            def inner(j, _):
                for b in range(DEP):
                    @pl.when((j + off) % DEP == b)
                    def _(j=j, b=b):
                        stepj(j, b)
                return 0

            lax.fori_loop(0, GCH - LOOK, inner, 0)

            for j in range(GCH - LOOK, GCH):
                b = (j + off) % DEP
                gwait(b)
                drain((b + LOOK) % DEP)
                if g < G - 1:
                    if j == GCH - LOOK:
                        pltpu.make_async_copy(
                            src_hbm.at[wid, pl.ds(0, GCH)], sbn, semi).wait()
                        pltpu.make_async_copy(
                            dst_hbm.at[wid, pl.ds(0, GCH)], dbn, semi).wait()
                    gissue(sbn.at[j - (GCH - LOOK)], (b + LOOK) % DEP)
                sissue(j, db, b)

        drain((G * GCH - 1) % DEP)
        plsc.subcore_barrier()

        ro = 0
        for sz in WBS:
            pltpu.sync_copy(acc.at[pl.ds(row0 + ro, sz)], r0.at[pl.ds(0, sz)])
            pltpu.sync_copy(r0.at[pl.ds(0, sz)],
                            out_hbm.at[cid, pl.ds(row0 + ro, sz)])
            ro += sz

    return k(h, src3, dst3)


def _tc_norms(deg_partials):
    def k(dp_ref, o_ref):
        deg = dp_ref[0] + dp_ref[1]
        o_ref[...] = lax.rsqrt(jnp.maximum(deg, 1.0))

    return pl.pallas_call(
        k,
        out_shape=jax.ShapeDtypeStruct((2, NPAD), jnp.float32),
    )(deg_partials)


def _tc_pre_matmul(x, ns, w):
    R = 1000

    def k(x_ref, ns_ref, w_ref, o_ref):
        o_ref[...] = jnp.dot(x_ref[...] * ns_ref[...], w_ref[...],
                             preferred_element_type=jnp.float32)

    d_in, d_out = w.shape
    return pl.pallas_call(
        k,
        grid=(N // R,),
        in_specs=[
            pl.BlockSpec((R, d_in), lambda i: (i, 0)),
            pl.BlockSpec((R, 1), lambda i: (i, 0)),
            pl.BlockSpec((d_in, d_out), lambda i: (0, 0)),
        ],
        out_specs=pl.BlockSpec((R, d_out), lambda i: (i, 0)),
        out_shape=jax.ShapeDtypeStruct((N, d_out), jnp.float32),
    )(x, ns, w)


def _tc_mid(partials, nd, ns, b, w):
    R = 1000

    def k(p_ref, nd_ref, ns_ref, b_ref, w_ref, o_ref):
        t = (p_ref[0] + p_ref[1]) * nd_ref[...] + b_ref[...]
        t = jnp.maximum(t, 0.0) * ns_ref[...]
        o_ref[...] = jnp.dot(t, w_ref[...], preferred_element_type=jnp.float32)

    d_in, d_out = w.shape
    return pl.pallas_call(
        k,
        grid=(N // R,),
        in_specs=[
            pl.BlockSpec((NC, R, d_in), lambda i: (0, i, 0)),
            pl.BlockSpec((R, 1), lambda i: (i, 0)),
            pl.BlockSpec((R, 1), lambda i: (i, 0)),
            pl.BlockSpec((1, d_in), lambda i: (0, 0)),
            pl.BlockSpec((d_in, d_out), lambda i: (0, 0)),
        ],
        out_specs=pl.BlockSpec((R, d_out), lambda i: (i, 0)),
        out_shape=jax.ShapeDtypeStruct((N, d_out), jnp.float32),
    )(partials, nd, ns, b, w)


def _tc_elem(partials, nd, ns, b):
    R = 1000

    def k(p_ref, nd_ref, ns_ref, b_ref, o_ref):
        t = (p_ref[0] + p_ref[1]) * nd_ref[...] + b_ref[...]
        o_ref[...] = jnp.maximum(t, 0.0) * ns_ref[...]

    return pl.pallas_call(
        k,
        grid=(N // R,),
        in_specs=[
            pl.BlockSpec((NC, R, D_H), lambda i: (0, i, 0)),
            pl.BlockSpec((R, 1), lambda i: (i, 0)),
            pl.BlockSpec((R, 1), lambda i: (i, 0)),
            pl.BlockSpec((1, D_H), lambda i: (0, 0)),
        ],
        out_specs=pl.BlockSpec((R, D_H), lambda i: (i, 0)),
        out_shape=jax.ShapeDtypeStruct((N, D_H), jnp.float32),
    )(partials, nd, ns, b)


def _tc_final_matmul(partials, nd, b, w):
    R = 1000

    def k(p_ref, nd_ref, b_ref, w_ref, o_ref):
        t = (p_ref[0] + p_ref[1]) * nd_ref[...]
        o_ref[...] = jnp.dot(t, w_ref[...],
                             preferred_element_type=jnp.float32) + b_ref[...]

    d_in, d_out = w.shape
    return pl.pallas_call(
        k,
        grid=(N // R,),
        in_specs=[
            pl.BlockSpec((NC, R, d_in), lambda i: (0, i, 0)),
            pl.BlockSpec((R, 1), lambda i: (i, 0)),
            pl.BlockSpec((1, d_out), lambda i: (0, 0)),
            pl.BlockSpec((d_in, d_out), lambda i: (0, 0)),
        ],
        out_specs=pl.BlockSpec((R, d_out), lambda i: (i, 0)),
        out_shape=jax.ShapeDtypeStruct((N, d_out), jnp.float32),
    )(partials, nd, b, w)


def kernel(x, edge_index, W1, b1, W2, b2, W3, b3):
    src = edge_index[0]
    dst = edge_index[1]

    deg_partials = _sc_degrees(src, dst)
    norms = _tc_norms(deg_partials)
    ns = norms[0, :N, None]
    nd = norms[1, :N, None]

    EPAD = NW * 80 * 128
    pad_ids = jnp.arange(EPAD - E, dtype=jnp.int32)
    src3 = jnp.concatenate(
        [src, pad_ids % N]).reshape(NW, 160, 64)
    dst3 = jnp.concatenate(
        [dst, N + pad_ids % (SPAD - N)]).reshape(NW, 160, 64)

    h = _tc_pre_matmul(x, ns, W1)
    p = _sc_spmm(h, src3, dst3, D_H)
    h = _tc_mid(p, nd, ns, b1[None, :], W2)
    p = _sc_spmm(h, src3, dst3, D_H)
    h = _tc_elem(p, nd, ns, b2[None, :])
    p = _sc_spmm(h, src3, dst3, D_H)
    return _tc_final_matmul(p, nd, b3[None, :], W3)
